# baseline (speedup 1.0000x reference)
import functools

import jax
import jax.numpy as jnp
from jax import lax
from jax.experimental import pallas as pl
from jax.experimental.pallas import tpu as pltpu
from jax.experimental.pallas import tpu_sc as plsc

N_ROWS = 1_000_000
N_CLS = 64


TC_BLOCK = 8000
TC_GRID = N_ROWS // TC_BLOCK


TC_CHUNK = 400


def _argmax_body(x_ref, o_ref):
    iota1 = lax.broadcasted_iota(jnp.int32, (1, N_CLS), 1).astype(jnp.float32)
    acc = jnp.zeros((1, N_CLS), jnp.float32)
    for ch in range(TC_BLOCK // TC_CHUNK):
        x = x_ref[pl.ds(ch * TC_CHUNK, TC_CHUNK), :]
        m = jnp.max(x, axis=1, keepdims=True)
        t = jnp.where(x == m, iota1, jnp.float32(N_CLS))
        fi = jnp.min(t, axis=1, keepdims=True)
        oh = jnp.where(iota1 == fi, jnp.float32(1.0), jnp.float32(0.0))
        acc = acc + jnp.sum(oh, axis=0, keepdims=True)
    o_ref[...] = acc.reshape(1, 1, N_CLS)


def _tc_counts(x):
    return pl.pallas_call(
        _argmax_body,
        grid=(TC_GRID,),
        in_specs=[pl.BlockSpec((TC_BLOCK, N_CLS), lambda i: (i, 0))],
        out_specs=pl.BlockSpec((1, 1, N_CLS), lambda i: (i, 0, 0)),
        out_shape=jax.ShapeDtypeStruct((TC_GRID, 1, N_CLS), jnp.float32),
    )(x)


SC_CH = 400
SC_GRP = SC_CH // 16
SC_CHW = SC_CH * N_CLS
SC_NCHUNKS = N_ROWS // SC_CH
SC_NTILES = 32


def _argcount_body(x_hbm, out_hbm, cb0, cb1, hist, outv, sem0, sem1):
    c = lax.axis_index("c")
    s = lax.axis_index("s")
    w = s * 2 + c

    lane = lax.broadcasted_iota(jnp.int32, (16,), 0)
    z16 = jnp.zeros((16,), jnp.int32)
    ones = jnp.ones((16,), jnp.int32)
    laneoff = lane * N_CLS
    lane64 = lane * N_CLS
    neginf = jnp.full((16,), -jnp.inf, jnp.float32)

    for i in range(N_CLS):
        hist[pl.ds(i * 16, 16)] = z16

    n_w = jnp.where(w < SC_NCHUNKS % SC_NTILES,
                    SC_NCHUNKS // SC_NTILES + 1, SC_NCHUNKS // SC_NTILES)

    def issue(k, buf, sem):
        base = (w + k * SC_NTILES) * SC_CHW
        pltpu.async_copy(x_hbm.at[pl.ds(base, SC_CHW)], buf, sem)

    def process(buf):
        def group(g, pb):
            m = neginf
            best = z16
            for j in range(N_CLS):
                v = plsc.load_gather(buf, [pb + j])
                gt = v > m
                m = jnp.maximum(m, v)
                best = jnp.where(gt, jnp.int32(j), best)
            plsc.addupdate_scatter(hist, [laneoff + best], ones)
            return pb + 16 * N_CLS

        lax.fori_loop(0, SC_GRP, group, lane64, unroll=4)

    issue(0, cb0, sem0)
    issue(1, cb1, sem1)

    def pair(k2, _):
        k = k2 * 2

        @pl.when(k < n_w)
        def _():
            pltpu.make_async_copy(x_hbm.at[pl.ds(0, SC_CHW)], cb0, sem0).wait()
            process(cb0)

            @pl.when(k + 2 < n_w)
            def _():
                issue(k + 2, cb0, sem0)

        @pl.when(k + 1 < n_w)
        def _():
            pltpu.make_async_copy(x_hbm.at[pl.ds(0, SC_CHW)], cb1, sem1).wait()
            process(cb1)

            @pl.when(k + 3 < n_w)
            def _():
                issue(k + 3, cb1, sem1)

        return 0

    lax.fori_loop(0, (SC_NCHUNKS // SC_NTILES + 2) // 2, pair, 0)

    for cg in range(4):
        acc = z16
        for r in range(16):
            acc = acc + hist[pl.ds(r * N_CLS + cg * 16, 16)]
        outv[pl.ds(cg * 16, 16)] = acc.astype(jnp.float32)
    pltpu.sync_copy(outv, out_hbm.at[pl.ds(w * N_CLS, N_CLS)])


def _sc_argcount(xflat):
    mesh = plsc.VectorSubcoreMesh(core_axis_name="c", subcore_axis_name="s")
    f = functools.partial(
        pl.kernel,
        mesh=mesh,
        compiler_params=pltpu.CompilerParams(needs_layout_passes=False),
        out_type=jax.ShapeDtypeStruct((SC_NTILES * N_CLS,), jnp.float32),
        scratch_types=[
            pltpu.VMEM((SC_CHW,), jnp.float32),
            pltpu.VMEM((SC_CHW,), jnp.float32),
            pltpu.VMEM((16 * N_CLS,), jnp.int32),
            pltpu.VMEM((N_CLS,), jnp.float32),
            pltpu.SemaphoreType.DMA,
            pltpu.SemaphoreType.DMA,
        ],
    )(_argcount_body)
    return f(xflat)


NSUB = 16
NPART = SC_NTILES * N_CLS
NPAD = 2048


def _reduce_body(p_hbm, out_hbm, pbuf, bins, iota_ref, outv, shared):
    c = lax.axis_index("c")
    s = lax.axis_index("s")
    active = c == 0

    lane = lax.broadcasted_iota(jnp.int32, (16,), 0)
    zf = jnp.zeros((16,), jnp.float32)

    for cg in range(4):
        iota_ref[pl.ds(cg * 16, 16)] = lane + cg * 16
    for cg in range(4):
        bins[pl.ds(cg * 16, 16)] = zf

    @pl.when(jnp.logical_and(active, s == 0))
    def _():
        pltpu.sync_copy(bins, shared)

    plsc.subcore_barrier()

    @pl.when(active)
    def _():
        pltpu.sync_copy(p_hbm, pbuf.at[pl.ds(0, NPART)])
        for i in range((NPAD - NPART) // 16):
            pbuf[pl.ds(NPART + i * 16, 16)] = zf
        for cg in range(4):
            acc = zf
            for k in range(NPAD // N_CLS // 16):
                acc = acc + pbuf[pl.ds((s + 16 * k) * N_CLS + cg * 16, 16)]
            bins[pl.ds(cg * 16, 16)] = acc
        pltpu.sync_copy(bins, shared.at[iota_ref], add=True)

    plsc.subcore_barrier()

    @pl.when(jnp.logical_and(active, s == 0))
    def _():
        pltpu.sync_copy(shared, bins)
        inv = jnp.float32(1.0 / N_ROWS)
        for cg in range(4):
            outv[pl.ds(cg * 16, 16)] = bins[pl.ds(cg * 16, 16)] * inv
        pltpu.sync_copy(outv, out_hbm)


def _sc_reduce(partials):
    mesh = plsc.VectorSubcoreMesh(core_axis_name="c", subcore_axis_name="s")
    f = functools.partial(
        pl.kernel,
        mesh=mesh,
        compiler_params=pltpu.CompilerParams(needs_layout_passes=False),
        out_type=jax.ShapeDtypeStruct((N_CLS,), jnp.float32),
        scratch_types=[
            pltpu.VMEM((NPAD,), jnp.float32),
            pltpu.VMEM((N_CLS,), jnp.float32),
            pltpu.VMEM((N_CLS,), jnp.int32),
            pltpu.VMEM((N_CLS,), jnp.float32),
            pltpu.VMEM_SHARED((N_CLS,), jnp.float32),
        ],
    )(_reduce_body)
    return f(partials)


def _probe_body(x_ref, o_ref):
    o_ref[...] = jnp.sum(x_ref[...], axis=0, keepdims=True).reshape(1, 1, N_CLS)


def _tc_probe(x):
    return pl.pallas_call(
        _probe_body,
        grid=(TC_GRID,),
        in_specs=[pl.BlockSpec((TC_BLOCK, N_CLS), lambda i: (i, 0))],
        out_specs=pl.BlockSpec((1, 1, N_CLS), lambda i: (i, 0, 0)),
        out_shape=jax.ShapeDtypeStruct((TC_GRID, 1, N_CLS), jnp.float32),
    )(x)


def kernel(input):
    partials = _sc_argcount(input.reshape(N_ROWS * N_CLS))
    probe = _tc_probe(input).reshape(TC_GRID * N_CLS)
    return _sc_reduce(partials) + probe[:N_CLS] * jnp.float32(1e-30)


# speedup vs baseline: 2.7995x; 2.7995x over previous
import functools

import jax
import jax.numpy as jnp
from jax import lax
from jax.experimental import pallas as pl
from jax.experimental.pallas import tpu as pltpu
from jax.experimental.pallas import tpu_sc as plsc

N_ROWS = 1_000_000
N_CLS = 64


TC_BLOCK = 8000
TC_GRID = N_ROWS // TC_BLOCK


TC_CHUNK = 2000


def _argmax_body(x_ref, o_ref):
    iota1 = lax.broadcasted_iota(jnp.int32, (1, N_CLS), 1).astype(jnp.float32)
    acc = jnp.zeros((1, N_CLS), jnp.float32)
    for ch in range(TC_BLOCK // TC_CHUNK):
        x = x_ref[pl.ds(ch * TC_CHUNK, TC_CHUNK), :]
        m = jnp.max(x, axis=1, keepdims=True)
        t = jnp.where(x == m, iota1, jnp.float32(N_CLS))
        fi = jnp.min(t, axis=1, keepdims=True)
        oh = jnp.where(iota1 == fi, jnp.float32(1.0), jnp.float32(0.0))
        acc = acc + jnp.sum(oh, axis=0, keepdims=True)
    o_ref[...] = acc.reshape(1, 1, N_CLS)


def _tc_counts(x):
    return pl.pallas_call(
        _argmax_body,
        grid=(TC_GRID,),
        in_specs=[pl.BlockSpec((TC_BLOCK, N_CLS), lambda i: (i, 0))],
        out_specs=pl.BlockSpec((1, 1, N_CLS), lambda i: (i, 0, 0)),
        out_shape=jax.ShapeDtypeStruct((TC_GRID, 1, N_CLS), jnp.float32),
    )(x)


SC_CH = 400
SC_GRP = SC_CH // 16
SC_CHW = SC_CH * N_CLS
SC_NCHUNKS = N_ROWS // SC_CH
SC_NTILES = 32


def _argcount_body(x_hbm, out_hbm, cb0, cb1, hist, outv, sem0, sem1):
    c = lax.axis_index("c")
    s = lax.axis_index("s")
    w = s * 2 + c

    lane = lax.broadcasted_iota(jnp.int32, (16,), 0)
    z16 = jnp.zeros((16,), jnp.int32)
    ones = jnp.ones((16,), jnp.int32)
    laneoff = lane * N_CLS
    lane64 = lane * N_CLS
    neginf = jnp.full((16,), -jnp.inf, jnp.float32)

    for i in range(N_CLS):
        hist[pl.ds(i * 16, 16)] = z16

    n_w = jnp.where(w < SC_NCHUNKS % SC_NTILES,
                    SC_NCHUNKS // SC_NTILES + 1, SC_NCHUNKS // SC_NTILES)

    def issue(k, buf, sem):
        base = (w + k * SC_NTILES) * SC_CHW
        pltpu.async_copy(x_hbm.at[pl.ds(base, SC_CHW)], buf, sem)

    def process(buf):
        def group(g, pb):
            m = neginf
            best = z16
            for j in range(N_CLS):
                v = plsc.load_gather(buf, [pb + j])
                gt = v > m
                m = jnp.maximum(m, v)
                best = jnp.where(gt, jnp.int32(j), best)
            plsc.addupdate_scatter(hist, [laneoff + best], ones)
            return pb + 16 * N_CLS

        lax.fori_loop(0, SC_GRP, group, lane64, unroll=4)

    issue(0, cb0, sem0)
    issue(1, cb1, sem1)

    def pair(k2, _):
        k = k2 * 2

        @pl.when(k < n_w)
        def _():
            pltpu.make_async_copy(x_hbm.at[pl.ds(0, SC_CHW)], cb0, sem0).wait()
            process(cb0)

            @pl.when(k + 2 < n_w)
            def _():
                issue(k + 2, cb0, sem0)

        @pl.when(k + 1 < n_w)
        def _():
            pltpu.make_async_copy(x_hbm.at[pl.ds(0, SC_CHW)], cb1, sem1).wait()
            process(cb1)

            @pl.when(k + 3 < n_w)
            def _():
                issue(k + 3, cb1, sem1)

        return 0

    lax.fori_loop(0, (SC_NCHUNKS // SC_NTILES + 2) // 2, pair, 0)

    for cg in range(4):
        acc = z16
        for r in range(16):
            acc = acc + hist[pl.ds(r * N_CLS + cg * 16, 16)]
        outv[pl.ds(cg * 16, 16)] = acc.astype(jnp.float32)
    pltpu.sync_copy(outv, out_hbm.at[pl.ds(w * N_CLS, N_CLS)])


def _sc_argcount(xflat):
    mesh = plsc.VectorSubcoreMesh(core_axis_name="c", subcore_axis_name="s")
    f = functools.partial(
        pl.kernel,
        mesh=mesh,
        compiler_params=pltpu.CompilerParams(needs_layout_passes=False),
        out_type=jax.ShapeDtypeStruct((SC_NTILES * N_CLS,), jnp.float32),
        scratch_types=[
            pltpu.VMEM((SC_CHW,), jnp.float32),
            pltpu.VMEM((SC_CHW,), jnp.float32),
            pltpu.VMEM((16 * N_CLS,), jnp.int32),
            pltpu.VMEM((N_CLS,), jnp.float32),
            pltpu.SemaphoreType.DMA,
            pltpu.SemaphoreType.DMA,
        ],
    )(_argcount_body)
    return f(xflat)


NSUB = 16
NPART = TC_GRID * N_CLS
NPAD = 8192


def _reduce_body(p_hbm, out_hbm, pbuf, bins, iota_ref, outv, shared):
    c = lax.axis_index("c")
    s = lax.axis_index("s")
    active = c == 0

    lane = lax.broadcasted_iota(jnp.int32, (16,), 0)
    zf = jnp.zeros((16,), jnp.float32)

    for cg in range(4):
        iota_ref[pl.ds(cg * 16, 16)] = lane + cg * 16
    for cg in range(4):
        bins[pl.ds(cg * 16, 16)] = zf

    @pl.when(jnp.logical_and(active, s == 0))
    def _():
        pltpu.sync_copy(bins, shared)

    plsc.subcore_barrier()

    @pl.when(active)
    def _():
        pltpu.sync_copy(p_hbm, pbuf.at[pl.ds(0, NPART)])
        for i in range((NPAD - NPART) // 16):
            pbuf[pl.ds(NPART + i * 16, 16)] = zf
        for cg in range(4):
            acc = zf
            for k in range(NPAD // N_CLS // 16):
                acc = acc + pbuf[pl.ds((s + 16 * k) * N_CLS + cg * 16, 16)]
            bins[pl.ds(cg * 16, 16)] = acc
        pltpu.sync_copy(bins, shared.at[iota_ref], add=True)

    plsc.subcore_barrier()

    @pl.when(jnp.logical_and(active, s == 0))
    def _():
        pltpu.sync_copy(shared, bins)
        inv = jnp.float32(1.0 / N_ROWS)
        for cg in range(4):
            outv[pl.ds(cg * 16, 16)] = bins[pl.ds(cg * 16, 16)] * inv
        pltpu.sync_copy(outv, out_hbm)


def _sc_reduce(partials):
    mesh = plsc.VectorSubcoreMesh(core_axis_name="c", subcore_axis_name="s")
    f = functools.partial(
        pl.kernel,
        mesh=mesh,
        compiler_params=pltpu.CompilerParams(needs_layout_passes=False),
        out_type=jax.ShapeDtypeStruct((N_CLS,), jnp.float32),
        scratch_types=[
            pltpu.VMEM((NPAD,), jnp.float32),
            pltpu.VMEM((N_CLS,), jnp.float32),
            pltpu.VMEM((N_CLS,), jnp.int32),
            pltpu.VMEM((N_CLS,), jnp.float32),
            pltpu.VMEM_SHARED((N_CLS,), jnp.float32),
        ],
    )(_reduce_body)
    return f(partials)


def kernel(input):
    partials = _tc_counts(input).reshape(NPART)
    return _sc_reduce(partials)


# speedup vs baseline: 3.0956x; 1.1058x over previous
import functools

import jax
import jax.numpy as jnp
from jax import lax
from jax.experimental import pallas as pl
from jax.experimental.pallas import tpu as pltpu
from jax.experimental.pallas import tpu_sc as plsc

N_ROWS = 1_000_000
N_CLS = 64


TC_BLOCK = 8000
TC_GRID = N_ROWS // TC_BLOCK


TC_CHUNK = 2000


def _argmax_body(x_ref, o_ref):
    ia = lax.broadcasted_iota(jnp.int32, (N_CLS, N_CLS), 0)
    ib = lax.broadcasted_iota(jnp.int32, (N_CLS, N_CLS), 1)
    u = jnp.where(ia < ib, jnp.float32(1.0), jnp.float32(0.0))
    acc = jnp.zeros((1, N_CLS), jnp.float32)
    for ch in range(TC_BLOCK // TC_CHUNK):
        x = x_ref[pl.ds(ch * TC_CHUNK, TC_CHUNK), :]
        m = jnp.max(x, axis=1, keepdims=True)
        eqf = jnp.where(x == m, jnp.float32(1.0), jnp.float32(0.0))
        ps = jax.lax.dot_general(eqf, u, (((1,), (0,)), ((), ())),
                                 preferred_element_type=jnp.float32)
        oh = jnp.where(ps == 0.0, eqf, jnp.float32(0.0))
        acc = acc + jnp.sum(oh, axis=0, keepdims=True)
    o_ref[...] = acc.reshape(1, 1, N_CLS)


def _tc_counts(x):
    return pl.pallas_call(
        _argmax_body,
        grid=(TC_GRID,),
        in_specs=[pl.BlockSpec((TC_BLOCK, N_CLS), lambda i: (i, 0))],
        out_specs=pl.BlockSpec((1, 1, N_CLS), lambda i: (i, 0, 0)),
        out_shape=jax.ShapeDtypeStruct((TC_GRID, 1, N_CLS), jnp.float32),
    )(x)


SC_CH = 400
SC_GRP = SC_CH // 16
SC_CHW = SC_CH * N_CLS
SC_NCHUNKS = N_ROWS // SC_CH
SC_NTILES = 32


def _argcount_body(x_hbm, out_hbm, cb0, cb1, hist, outv, sem0, sem1):
    c = lax.axis_index("c")
    s = lax.axis_index("s")
    w = s * 2 + c

    lane = lax.broadcasted_iota(jnp.int32, (16,), 0)
    z16 = jnp.zeros((16,), jnp.int32)
    ones = jnp.ones((16,), jnp.int32)
    laneoff = lane * N_CLS
    lane64 = lane * N_CLS
    neginf = jnp.full((16,), -jnp.inf, jnp.float32)

    for i in range(N_CLS):
        hist[pl.ds(i * 16, 16)] = z16

    n_w = jnp.where(w < SC_NCHUNKS % SC_NTILES,
                    SC_NCHUNKS // SC_NTILES + 1, SC_NCHUNKS // SC_NTILES)

    def issue(k, buf, sem):
        base = (w + k * SC_NTILES) * SC_CHW
        pltpu.async_copy(x_hbm.at[pl.ds(base, SC_CHW)], buf, sem)

    def process(buf):
        def group(g, pb):
            m = neginf
            best = z16
            for j in range(N_CLS):
                v = plsc.load_gather(buf, [pb + j])
                gt = v > m
                m = jnp.maximum(m, v)
                best = jnp.where(gt, jnp.int32(j), best)
            plsc.addupdate_scatter(hist, [laneoff + best], ones)
            return pb + 16 * N_CLS

        lax.fori_loop(0, SC_GRP, group, lane64, unroll=4)

    issue(0, cb0, sem0)
    issue(1, cb1, sem1)

    def pair(k2, _):
        k = k2 * 2

        @pl.when(k < n_w)
        def _():
            pltpu.make_async_copy(x_hbm.at[pl.ds(0, SC_CHW)], cb0, sem0).wait()
            process(cb0)

            @pl.when(k + 2 < n_w)
            def _():
                issue(k + 2, cb0, sem0)

        @pl.when(k + 1 < n_w)
        def _():
            pltpu.make_async_copy(x_hbm.at[pl.ds(0, SC_CHW)], cb1, sem1).wait()
            process(cb1)

            @pl.when(k + 3 < n_w)
            def _():
                issue(k + 3, cb1, sem1)

        return 0

    lax.fori_loop(0, (SC_NCHUNKS // SC_NTILES + 2) // 2, pair, 0)

    for cg in range(4):
        acc = z16
        for r in range(16):
            acc = acc + hist[pl.ds(r * N_CLS + cg * 16, 16)]
        outv[pl.ds(cg * 16, 16)] = acc.astype(jnp.float32)
    pltpu.sync_copy(outv, out_hbm.at[pl.ds(w * N_CLS, N_CLS)])


def _sc_argcount(xflat):
    mesh = plsc.VectorSubcoreMesh(core_axis_name="c", subcore_axis_name="s")
    f = functools.partial(
        pl.kernel,
        mesh=mesh,
        compiler_params=pltpu.CompilerParams(needs_layout_passes=False),
        out_type=jax.ShapeDtypeStruct((SC_NTILES * N_CLS,), jnp.float32),
        scratch_types=[
            pltpu.VMEM((SC_CHW,), jnp.float32),
            pltpu.VMEM((SC_CHW,), jnp.float32),
            pltpu.VMEM((16 * N_CLS,), jnp.int32),
            pltpu.VMEM((N_CLS,), jnp.float32),
            pltpu.SemaphoreType.DMA,
            pltpu.SemaphoreType.DMA,
        ],
    )(_argcount_body)
    return f(xflat)


NSUB = 16
NPART = TC_GRID * N_CLS
NPAD = 8192


def _reduce_body(p_hbm, out_hbm, pbuf, bins, iota_ref, outv, shared):
    c = lax.axis_index("c")
    s = lax.axis_index("s")
    active = c == 0

    lane = lax.broadcasted_iota(jnp.int32, (16,), 0)
    zf = jnp.zeros((16,), jnp.float32)

    for cg in range(4):
        iota_ref[pl.ds(cg * 16, 16)] = lane + cg * 16
    for cg in range(4):
        bins[pl.ds(cg * 16, 16)] = zf

    @pl.when(jnp.logical_and(active, s == 0))
    def _():
        pltpu.sync_copy(bins, shared)

    plsc.subcore_barrier()

    @pl.when(active)
    def _():
        pltpu.sync_copy(p_hbm, pbuf.at[pl.ds(0, NPART)])
        for i in range((NPAD - NPART) // 16):
            pbuf[pl.ds(NPART + i * 16, 16)] = zf
        for cg in range(4):
            acc = zf
            for k in range(NPAD // N_CLS // 16):
                acc = acc + pbuf[pl.ds((s + 16 * k) * N_CLS + cg * 16, 16)]
            bins[pl.ds(cg * 16, 16)] = acc
        pltpu.sync_copy(bins, shared.at[iota_ref], add=True)

    plsc.subcore_barrier()

    @pl.when(jnp.logical_and(active, s == 0))
    def _():
        pltpu.sync_copy(shared, bins)
        inv = jnp.float32(1.0 / N_ROWS)
        for cg in range(4):
            outv[pl.ds(cg * 16, 16)] = bins[pl.ds(cg * 16, 16)] * inv
        pltpu.sync_copy(outv, out_hbm)


def _sc_reduce(partials):
    mesh = plsc.VectorSubcoreMesh(core_axis_name="c", subcore_axis_name="s")
    f = functools.partial(
        pl.kernel,
        mesh=mesh,
        compiler_params=pltpu.CompilerParams(needs_layout_passes=False),
        out_type=jax.ShapeDtypeStruct((N_CLS,), jnp.float32),
        scratch_types=[
            pltpu.VMEM((NPAD,), jnp.float32),
            pltpu.VMEM((N_CLS,), jnp.float32),
            pltpu.VMEM((N_CLS,), jnp.int32),
            pltpu.VMEM((N_CLS,), jnp.float32),
            pltpu.VMEM_SHARED((N_CLS,), jnp.float32),
        ],
    )(_reduce_body)
    return f(partials)


def kernel(input):
    partials = _tc_counts(input).reshape(NPART)
    return _sc_reduce(partials)


# speedup vs baseline: 3.0973x; 1.0005x over previous
import functools

import jax
import jax.numpy as jnp
from jax import lax
from jax.experimental import pallas as pl
from jax.experimental.pallas import tpu as pltpu
from jax.experimental.pallas import tpu_sc as plsc

N_ROWS = 1_000_000
N_CLS = 64


TC_BLOCK = 8000
TC_GRID = N_ROWS // TC_BLOCK


TC_CHUNK = 2000


def _argmax_body(x_ref, o_ref):
    ia = lax.broadcasted_iota(jnp.int32, (N_CLS, N_CLS), 0)
    ib = lax.broadcasted_iota(jnp.int32, (N_CLS, N_CLS), 1)
    u = jnp.where(ia < ib, jnp.float32(1.0), jnp.float32(0.0))
    acc = jnp.zeros((1, N_CLS), jnp.float32)
    for ch in range(TC_BLOCK // TC_CHUNK):
        x = x_ref[pl.ds(ch * TC_CHUNK, TC_CHUNK), :]
        m = jnp.max(x, axis=1, keepdims=True)
        eqf = jnp.where(x == m, jnp.float32(1.0), jnp.float32(0.0))
        ps = jax.lax.dot_general(eqf, u, (((1,), (0,)), ((), ())),
                                 preferred_element_type=jnp.float32)
        oh = jnp.where(ps == 0.0, eqf, jnp.float32(0.0))
        acc = acc + jnp.sum(oh, axis=0, keepdims=True)
    o_ref[...] = acc.reshape(1, 1, N_CLS)


def _tc_counts(x):
    return pl.pallas_call(
        _argmax_body,
        grid=(TC_GRID,),
        in_specs=[pl.BlockSpec((TC_BLOCK, N_CLS), lambda i: (i, 0))],
        out_specs=pl.BlockSpec((1, 1, N_CLS), lambda i: (i, 0, 0)),
        out_shape=jax.ShapeDtypeStruct((TC_GRID, 1, N_CLS), jnp.float32),
        compiler_params=pltpu.CompilerParams(
            dimension_semantics=("arbitrary",)),
    )(x)


SC_CH = 400
SC_GRP = SC_CH // 16
SC_CHW = SC_CH * N_CLS
SC_NCHUNKS = N_ROWS // SC_CH
SC_NTILES = 32


def _argcount_body(x_hbm, out_hbm, cb0, cb1, hist, outv, sem0, sem1):
    c = lax.axis_index("c")
    s = lax.axis_index("s")
    w = s * 2 + c

    lane = lax.broadcasted_iota(jnp.int32, (16,), 0)
    z16 = jnp.zeros((16,), jnp.int32)
    ones = jnp.ones((16,), jnp.int32)
    laneoff = lane * N_CLS
    lane64 = lane * N_CLS
    neginf = jnp.full((16,), -jnp.inf, jnp.float32)

    for i in range(N_CLS):
        hist[pl.ds(i * 16, 16)] = z16

    n_w = jnp.where(w < SC_NCHUNKS % SC_NTILES,
                    SC_NCHUNKS // SC_NTILES + 1, SC_NCHUNKS // SC_NTILES)

    def issue(k, buf, sem):
        base = (w + k * SC_NTILES) * SC_CHW
        pltpu.async_copy(x_hbm.at[pl.ds(base, SC_CHW)], buf, sem)

    def process(buf):
        def group(g, pb):
            m = neginf
            best = z16
            for j in range(N_CLS):
                v = plsc.load_gather(buf, [pb + j])
                gt = v > m
                m = jnp.maximum(m, v)
                best = jnp.where(gt, jnp.int32(j), best)
            plsc.addupdate_scatter(hist, [laneoff + best], ones)
            return pb + 16 * N_CLS

        lax.fori_loop(0, SC_GRP, group, lane64, unroll=4)

    issue(0, cb0, sem0)
    issue(1, cb1, sem1)

    def pair(k2, _):
        k = k2 * 2

        @pl.when(k < n_w)
        def _():
            pltpu.make_async_copy(x_hbm.at[pl.ds(0, SC_CHW)], cb0, sem0).wait()
            process(cb0)

            @pl.when(k + 2 < n_w)
            def _():
                issue(k + 2, cb0, sem0)

        @pl.when(k + 1 < n_w)
        def _():
            pltpu.make_async_copy(x_hbm.at[pl.ds(0, SC_CHW)], cb1, sem1).wait()
            process(cb1)

            @pl.when(k + 3 < n_w)
            def _():
                issue(k + 3, cb1, sem1)

        return 0

    lax.fori_loop(0, (SC_NCHUNKS // SC_NTILES + 2) // 2, pair, 0)

    for cg in range(4):
        acc = z16
        for r in range(16):
            acc = acc + hist[pl.ds(r * N_CLS + cg * 16, 16)]
        outv[pl.ds(cg * 16, 16)] = acc.astype(jnp.float32)
    pltpu.sync_copy(outv, out_hbm.at[pl.ds(w * N_CLS, N_CLS)])


def _sc_argcount(xflat):
    mesh = plsc.VectorSubcoreMesh(core_axis_name="c", subcore_axis_name="s")
    f = functools.partial(
        pl.kernel,
        mesh=mesh,
        compiler_params=pltpu.CompilerParams(needs_layout_passes=False),
        out_type=jax.ShapeDtypeStruct((SC_NTILES * N_CLS,), jnp.float32),
        scratch_types=[
            pltpu.VMEM((SC_CHW,), jnp.float32),
            pltpu.VMEM((SC_CHW,), jnp.float32),
            pltpu.VMEM((16 * N_CLS,), jnp.int32),
            pltpu.VMEM((N_CLS,), jnp.float32),
            pltpu.SemaphoreType.DMA,
            pltpu.SemaphoreType.DMA,
        ],
    )(_argcount_body)
    return f(xflat)


NSUB = 16
NPART = TC_GRID * N_CLS
NPAD = 8192


def _reduce_body(p_hbm, out_hbm, pbuf, bins, iota_ref, outv, shared):
    c = lax.axis_index("c")
    s = lax.axis_index("s")
    active = c == 0

    lane = lax.broadcasted_iota(jnp.int32, (16,), 0)
    zf = jnp.zeros((16,), jnp.float32)

    for cg in range(4):
        iota_ref[pl.ds(cg * 16, 16)] = lane + cg * 16
    for cg in range(4):
        bins[pl.ds(cg * 16, 16)] = zf

    @pl.when(jnp.logical_and(active, s == 0))
    def _():
        pltpu.sync_copy(bins, shared)

    plsc.subcore_barrier()

    @pl.when(active)
    def _():
        pltpu.sync_copy(p_hbm, pbuf.at[pl.ds(0, NPART)])
        for i in range((NPAD - NPART) // 16):
            pbuf[pl.ds(NPART + i * 16, 16)] = zf
        for cg in range(4):
            acc = zf
            for k in range(NPAD // N_CLS // 16):
                acc = acc + pbuf[pl.ds((s + 16 * k) * N_CLS + cg * 16, 16)]
            bins[pl.ds(cg * 16, 16)] = acc
        pltpu.sync_copy(bins, shared.at[iota_ref], add=True)

    plsc.subcore_barrier()

    @pl.when(jnp.logical_and(active, s == 0))
    def _():
        pltpu.sync_copy(shared, bins)
        inv = jnp.float32(1.0 / N_ROWS)
        for cg in range(4):
            outv[pl.ds(cg * 16, 16)] = bins[pl.ds(cg * 16, 16)] * inv
        pltpu.sync_copy(outv, out_hbm)


def _sc_reduce(partials):
    mesh = plsc.VectorSubcoreMesh(core_axis_name="c", subcore_axis_name="s")
    f = functools.partial(
        pl.kernel,
        mesh=mesh,
        compiler_params=pltpu.CompilerParams(needs_layout_passes=False),
        out_type=jax.ShapeDtypeStruct((N_CLS,), jnp.float32),
        scratch_types=[
            pltpu.VMEM((NPAD,), jnp.float32),
            pltpu.VMEM((N_CLS,), jnp.float32),
            pltpu.VMEM((N_CLS,), jnp.int32),
            pltpu.VMEM((N_CLS,), jnp.float32),
            pltpu.VMEM_SHARED((N_CLS,), jnp.float32),
        ],
    )(_reduce_body)
    return f(partials)


def kernel(input):
    partials = _tc_counts(input).reshape(NPART)
    return _sc_reduce(partials)
